# baseline (speedup 1.0000x reference)
import functools

import jax
import jax.numpy as jnp
from jax import lax
from jax.experimental import pallas as pl
from jax.experimental.pallas import tpu as pltpu
from jax.experimental.pallas import tpu_sc as plsc

NC = 2
NS = 16
NW = NC * NS
D = 1024
C = 32
NBUF = 3


@functools.lru_cache(maxsize=None)
def _make(B):
    bpw = B // NW
    nchunks = bpw // C
    mesh = plsc.VectorSubcoreMesh(core_axis_name="c", subcore_axis_name="s")

    @functools.partial(
        pl.kernel,
        mesh=mesh,
        out_type=jax.ShapeDtypeStruct((B, D), jnp.float32),
        scratch_types=[
            pltpu.VMEM((bpw,), jnp.int32),
            pltpu.VMEM((NBUF, C, D), jnp.float32),
            pltpu.SemaphoreType.DMA((NBUF,)),
            pltpu.SemaphoreType.DMA((NBUF,)),
        ],
    )
    def gather_kernel(table_hbm, idx_hbm, out_hbm, idx_v, rows_v, gsem, ssem):
        wid = lax.axis_index("s") * NC + lax.axis_index("c")
        base = wid * bpw
        pltpu.sync_copy(idx_hbm.at[pl.ds(base, bpw)], idx_v)

        def gather(i):
            return pltpu.make_async_copy(
                table_hbm.at[idx_v.at[pl.ds(i * C, C)]],
                rows_v.at[lax.rem(i, NBUF)],
                gsem.at[lax.rem(i, NBUF)],
            )

        def writeback(i):
            return pltpu.make_async_copy(
                rows_v.at[lax.rem(i, NBUF)],
                out_hbm.at[pl.ds(base + i * C, C)],
                ssem.at[lax.rem(i, NBUF)],
            )

        for j in range(NBUF - 1):
            gather(j).start()

        def chunk(i, carry):
            @pl.when(i + NBUF - 1 < nchunks)
            def _prefetch():
                @pl.when(i >= 1)
                def _():
                    writeback(i - 1).wait()

                gather(i + NBUF - 1).start()

            gather(i).wait()
            writeback(i).start()
            return carry

        lax.fori_loop(0, nchunks, chunk, 0, unroll=False)
        for j in range(NBUF):
            writeback(nchunks - NBUF + j).wait()

    return gather_kernel


def kernel(input_batch, table):
    shape = input_batch.shape
    idx = input_batch.reshape(-1).astype(jnp.int32)
    out = _make(idx.shape[0])(table, idx)
    return out.reshape(*shape, D)
